# baseline (speedup 1.0000x reference)
import jax
import jax.numpy as jnp
from jax.experimental import pallas as pl
from jax.experimental.pallas import tpu as pltpu

_MAX_LEN = 200
_EMBED_DIM = 64
_FLAT = _MAX_LEN * _EMBED_DIM
_BATCH = 4096
_BB = 256
_HALF = _BATCH // 2
_NBLK = _HALF // _BB


def _alloc_body(o_ref):
    pass


def _main_body(pe_ref, buf_hbm, o_ref, scratch, sems):
    i = pl.program_id(0)
    o_ref[...] = jnp.broadcast_to(pe_ref[...], o_ref.shape)

    @pl.when(i == 0)
    def _():
        scratch[...] = jnp.broadcast_to(pe_ref[...], scratch.shape)
        for j in range(_NBLK):
            pltpu.make_async_copy(
                scratch,
                buf_hbm.at[pl.ds(_HALF + j * _BB, _BB), :],
                sems.at[j],
            ).start()

    @pl.when(i == _NBLK - 1)
    def _():
        for j in range(_NBLK):
            pltpu.make_async_copy(
                scratch,
                buf_hbm.at[pl.ds(_HALF + j * _BB, _BB), :],
                sems.at[j],
            ).wait()


def kernel(x, pe_weight):
    batch = x.shape[0]
    pe_flat = pe_weight.reshape(1, _FLAT)
    buf = pl.pallas_call(
        _alloc_body,
        out_specs=pl.BlockSpec(memory_space=pltpu.MemorySpace.HBM),
        out_shape=jax.ShapeDtypeStruct((batch, _FLAT), jnp.float32),
    )()
    out = pl.pallas_call(
        _main_body,
        grid=(_NBLK,),
        in_specs=[
            pl.BlockSpec((1, _FLAT), lambda i: (0, 0)),
            pl.BlockSpec(memory_space=pltpu.MemorySpace.HBM),
        ],
        out_specs=pl.BlockSpec((_BB, _FLAT), lambda i: (i, 0)),
        out_shape=jax.ShapeDtypeStruct((batch, _FLAT), jnp.float32),
        scratch_shapes=[
            pltpu.VMEM((_BB, _FLAT), jnp.float32),
            pltpu.SemaphoreType.DMA((_NBLK,)),
        ],
        input_output_aliases={1: 0},
    )(pe_flat, buf)
    return out.reshape(batch, _MAX_LEN, _EMBED_DIM)
